# Initial kernel scaffold; baseline (speedup 1.0000x reference)
import jax
import jax.numpy as jnp
from jax.experimental import pallas as pl


def kernel(x, conv1_w, conv1_scale, conv1_bias, conv2_w, conv2_scale, conv2_bias, conv3_w, conv3_scale, conv3_bias, conv4_w, conv4_scale, conv4_bias, conv5_w, conv5_scale, conv5_bias, fc_w, fc_b):
    raise NotImplementedError("write your pallas kernel here")


# speedup vs baseline: 2.0608x; 2.0608x over previous
import functools

import jax
import jax.numpy as jnp
import numpy as np
from jax.experimental import pallas as pl
from jax.experimental.pallas import tpu as pltpu

_HAAR = 0.5 * np.array(
    [[1, 1, 1, 1],
     [-1, -1, 1, 1],
     [-1, 1, -1, 1],
     [1, -1, -1, 1]], np.float32)


def _fold_haar(w, c_quarter):
    a = np.kron(_HAAR.T, np.eye(c_quarter, dtype=np.float32))
    w9 = w.astype(jnp.float32).reshape(9, 4 * c_quarter, -1)
    return jnp.einsum("pk,tko->tpo", jnp.asarray(a), w9).astype(jnp.bfloat16)


def _deint_pad(y, hh, wh):
    n, _, _, c = y.shape
    yr = y.reshape(n, hh, 2, wh, 2, c)
    p = jnp.concatenate(
        [yr[:, :, 0, :, 0, :], yr[:, :, 1, :, 0, :],
         yr[:, :, 0, :, 1, :], yr[:, :, 1, :, 1, :]], axis=-1)
    p = jnp.pad(p.astype(jnp.bfloat16), ((0, 0), (1, 1), (1, 1), (0, 0)))
    p = p.reshape(n, (hh + 2) * (wh + 2), 4 * c)
    return jnp.pad(p, ((0, 0), (1, 1), (0, 0)))


def _conv_body(h, wp, x_ref, w_ref, s_ref, b_ref, o_ref):
    mv = h * wp
    acc = None
    for dy in range(3):
        for dx in range(3):
            t = dy * wp + dx
            part = jnp.dot(x_ref[0, t:t + mv, :], w_ref[dy * 3 + dx],
                           preferred_element_type=jnp.float32)
            acc = part if acc is None else acc + part
    o_ref[0] = jnp.maximum(acc * s_ref[...] + b_ref[...], 0.0).astype(o_ref.dtype)


def _conv_stage(xf, w9, s, b, h, wp):
    n, l, cin = xf.shape
    cout = w9.shape[-1]
    mv = h * wp
    return pl.pallas_call(
        functools.partial(_conv_body, h, wp),
        out_shape=jax.ShapeDtypeStruct((n, mv, cout), jnp.bfloat16),
        grid=(n,),
        in_specs=[pl.BlockSpec((1, l, cin), lambda i: (i, 0, 0)),
                  pl.BlockSpec((9, cin, cout), lambda i: (0, 0, 0)),
                  pl.BlockSpec((1, cout), lambda i: (0, 0)),
                  pl.BlockSpec((1, cout), lambda i: (0, 0))],
        out_specs=pl.BlockSpec((1, mv, cout), lambda i: (i, 0, 0)),
        compiler_params=pltpu.CompilerParams(
            dimension_semantics=("parallel",)),
    )(xf, w9, s, b)


def _tail_body(x_ref, w4_ref, s4_ref, b4_ref, w5_ref, s5_ref, b5_ref,
               fw_ref, fb_ref, o_ref, scr_ref):
    wp, mv = 18, 288
    col = jax.lax.broadcasted_iota(jnp.int32, (mv, 1), 0) % wp
    interior = jnp.logical_and(col >= 1, col <= 16)

    acc = None
    for tap in range(9):
        t = (tap // 3) * wp + (tap % 3)
        part = jnp.dot(x_ref[0, t:t + mv, :], w4_ref[tap],
                       preferred_element_type=jnp.float32)
        acc = part if acc is None else acc + part
    y4 = jnp.maximum(acc * s4_ref[...] + b4_ref[...], 0.0)
    y4 = jnp.where(interior, y4, 0.0).astype(jnp.bfloat16)

    scr_ref[0:19, :] = jnp.zeros((19, 128), jnp.bfloat16)
    scr_ref[307:326, :] = jnp.zeros((19, 128), jnp.bfloat16)
    scr_ref[19:307, :] = y4

    acc5 = None
    for tap in range(9):
        t = (tap // 3) * wp + (tap % 3)
        part = jnp.dot(scr_ref[t:t + mv, :], w5_ref[tap],
                       preferred_element_type=jnp.float32)
        acc5 = part if acc5 is None else acc5 + part
    y5 = jnp.maximum(acc5 * s5_ref[...] + b5_ref[...], 0.0)
    y5 = jnp.where(interior, y5, 0.0)

    pooled = jnp.sum(y5, axis=0, keepdims=True) * (1.0 / 256.0)
    z = jnp.dot(pooled, fw_ref[...], preferred_element_type=jnp.float32)
    z = z + fb_ref[...]
    o_ref[0] = 1.0 / (1.0 + jnp.exp(-z))


def _tail_stage(xf, w4, s4, b4, w5, s5, b5, fw, fb):
    n, l, cin = xf.shape
    out = pl.pallas_call(
        _tail_body,
        out_shape=jax.ShapeDtypeStruct((n, 1, 1), jnp.float32),
        grid=(n,),
        in_specs=[pl.BlockSpec((1, l, cin), lambda i: (i, 0, 0)),
                  pl.BlockSpec((9, cin, 128), lambda i: (0, 0, 0)),
                  pl.BlockSpec((1, 128), lambda i: (0, 0)),
                  pl.BlockSpec((1, 128), lambda i: (0, 0)),
                  pl.BlockSpec((9, 128, 128), lambda i: (0, 0, 0)),
                  pl.BlockSpec((1, 128), lambda i: (0, 0)),
                  pl.BlockSpec((1, 128), lambda i: (0, 0)),
                  pl.BlockSpec((128, 1), lambda i: (0, 0)),
                  pl.BlockSpec((1, 1), lambda i: (0, 0))],
        out_specs=pl.BlockSpec((1, 1, 1), lambda i: (i, 0, 0)),
        scratch_shapes=[pltpu.VMEM((326, 128), jnp.bfloat16)],
        compiler_params=pltpu.CompilerParams(
            dimension_semantics=("parallel",)),
    )(xf, w4, s4, b4, w5, s5, b5, fw, fb)
    return out.reshape(n, 1)


def kernel(x,
           conv1_w, conv1_scale, conv1_bias,
           conv2_w, conv2_scale, conv2_bias,
           conv3_w, conv3_scale, conv3_bias,
           conv4_w, conv4_scale, conv4_bias,
           conv5_w, conv5_scale, conv5_bias,
           fc_w, fc_b):
    n = x.shape[0]
    w1 = _fold_haar(conv1_w, 4)
    w2 = _fold_haar(conv2_w, 16)
    w3 = _fold_haar(conv3_w, 64)
    w4 = _fold_haar(conv4_w, 128)
    w5 = conv5_w.astype(jnp.bfloat16).reshape(9, 128, 128)
    r = lambda v: v.astype(jnp.float32).reshape(1, -1)

    xf = _deint_pad(x, 128, 128)
    y1 = _conv_stage(xf, w1, r(conv1_scale), r(conv1_bias), 128, 130)
    xf = _deint_pad(y1.reshape(n, 128, 130, 16)[:, :, 1:129, :], 64, 64)
    y2 = _conv_stage(xf, w2, r(conv2_scale), r(conv2_bias), 64, 66)
    xf = _deint_pad(y2.reshape(n, 64, 66, 64)[:, :, 1:65, :], 32, 32)
    y3 = _conv_stage(xf, w3, r(conv3_scale), r(conv3_bias), 32, 34)
    xf = _deint_pad(y3.reshape(n, 32, 34, 128)[:, :, 1:33, :], 16, 16)
    return _tail_stage(xf, w4, r(conv4_scale), r(conv4_bias),
                       w5, r(conv5_scale), r(conv5_bias),
                       fc_w.astype(jnp.float32).reshape(128, 1),
                       fc_b.astype(jnp.float32).reshape(1, 1))


# speedup vs baseline: 2.3812x; 1.1555x over previous
import functools

import jax
import jax.numpy as jnp
import numpy as np
from jax.experimental import pallas as pl
from jax.experimental.pallas import tpu as pltpu

_HAAR = 0.5 * np.array(
    [[1, 1, 1, 1],
     [-1, -1, 1, 1],
     [-1, 1, -1, 1],
     [1, -1, -1, 1]], np.float32)


def _fold_haar(w, c_quarter):
    a = np.kron(_HAAR.T, np.eye(c_quarter, dtype=np.float32))
    w9 = w.astype(jnp.float32).reshape(9, 4 * c_quarter, -1)
    return jnp.einsum("pk,tko->tpo", jnp.asarray(a), w9)


def _pack_w(w9, p):
    _, k, co = w9.shape
    w33 = w9.reshape(3, 3, k, co)
    ui = np.arange(p)[:, None]
    uo = np.arange(p)[None, :]
    taps = []
    for dy in range(3):
        for gd in range(3):
            delta = p * (gd - 1) + ui - uo
            sel = np.clip(delta + 1, 0, 2)
            msk = jnp.asarray((np.abs(delta) <= 1).astype(np.float32))
            g = w33[dy][sel] * msk[:, :, None, None]
            taps.append(g.transpose(0, 2, 1, 3).reshape(p * k, p * co))
    return jnp.stack(taps).astype(jnp.bfloat16)


def _deint_pack_pad(y, hh, wh, p):
    n, _, _, c = y.shape
    yr = y.reshape(n, hh, 2, wh, 2, c)
    q = jnp.concatenate(
        [yr[:, :, 0, :, 0, :], yr[:, :, 1, :, 0, :],
         yr[:, :, 0, :, 1, :], yr[:, :, 1, :, 1, :]], axis=-1)
    q = q.astype(jnp.bfloat16).reshape(n, hh, wh // p, 4 * c * p)
    q = jnp.pad(q, ((0, 0), (1, 1), (1, 1), (0, 0)))
    q = q.reshape(n, (hh + 2) * (wh // p + 2), 4 * c * p)
    return jnp.pad(q, ((0, 0), (1, 1), (0, 0)))


def _unpack(o, h, wgp, p, co):
    n = o.shape[0]
    o = o.reshape(n, h, wgp, p, co)[:, :, 1:wgp - 1]
    return o.reshape(n, h, (wgp - 2) * p, co)


def _conv_body(h, wp, mt, x_ref, w_ref, s_ref, b_ref, o_ref):
    mv = h * wp
    for m0 in range(0, mv, mt):
        mb = min(mt, mv - m0)
        acc = None
        for dy in range(3):
            for dx in range(3):
                t = dy * wp + dx + m0
                part = jnp.dot(x_ref[0, t:t + mb, :], w_ref[dy * 3 + dx],
                               preferred_element_type=jnp.float32)
                acc = part if acc is None else acc + part
        y = jnp.maximum(acc * s_ref[...] + b_ref[...], 0.0)
        o_ref[0, m0:m0 + mb, :] = y.astype(o_ref.dtype)


def _conv_stage(xf, w9, s, b, h, wp, mt):
    n, l, cin = xf.shape
    cout = w9.shape[-1]
    mv = h * wp
    return pl.pallas_call(
        functools.partial(_conv_body, h, wp, mt),
        out_shape=jax.ShapeDtypeStruct((n, mv, cout), jnp.bfloat16),
        grid=(n,),
        in_specs=[pl.BlockSpec((1, l, cin), lambda i: (i, 0, 0)),
                  pl.BlockSpec((9, cin, cout), lambda i: (0, 0, 0)),
                  pl.BlockSpec((1, cout), lambda i: (0, 0)),
                  pl.BlockSpec((1, cout), lambda i: (0, 0))],
        out_specs=pl.BlockSpec((1, mv, cout), lambda i: (i, 0, 0)),
        compiler_params=pltpu.CompilerParams(
            dimension_semantics=("parallel",)),
    )(xf, w9, s, b)


def _tail_body(x_ref, w4_ref, s4_ref, b4_ref, w5_ref, s5_ref, b5_ref,
               fw_ref, fb_ref, o_ref, scr_ref):
    wp, mv = 10, 160
    col = jax.lax.broadcasted_iota(jnp.int32, (mv, 1), 0) % wp
    interior = jnp.logical_and(col >= 1, col <= 8)

    acc = None
    for tap in range(9):
        t = (tap // 3) * wp + (tap % 3)
        part = jnp.dot(x_ref[0, t:t + mv, :], w4_ref[tap],
                       preferred_element_type=jnp.float32)
        acc = part if acc is None else acc + part
    y4 = jnp.maximum(acc * s4_ref[...] + b4_ref[...], 0.0)
    y4 = jnp.where(interior, y4, 0.0).astype(jnp.bfloat16)

    scr_ref[0:11, :] = jnp.zeros((11, 256), jnp.bfloat16)
    scr_ref[171:182, :] = jnp.zeros((11, 256), jnp.bfloat16)
    scr_ref[11:171, :] = y4

    acc5 = None
    for tap in range(9):
        t = (tap // 3) * wp + (tap % 3)
        part = jnp.dot(scr_ref[t:t + mv, :], w5_ref[tap],
                       preferred_element_type=jnp.float32)
        acc5 = part if acc5 is None else acc5 + part
    y5 = jnp.maximum(acc5 * s5_ref[...] + b5_ref[...], 0.0)
    y5 = jnp.where(interior, y5, 0.0)

    pooled = jnp.sum(y5, axis=0, keepdims=True) * (1.0 / 256.0)
    pooled = pooled[:, 0:128] + pooled[:, 128:256]
    z = jnp.dot(pooled, fw_ref[...], preferred_element_type=jnp.float32)
    z = z + fb_ref[...]
    o_ref[0] = 1.0 / (1.0 + jnp.exp(-z))


def _tail_stage(xf, w4, s4, b4, w5, s5, b5, fw, fb):
    n, l, cin = xf.shape
    out = pl.pallas_call(
        _tail_body,
        out_shape=jax.ShapeDtypeStruct((n, 1, 1), jnp.float32),
        grid=(n,),
        in_specs=[pl.BlockSpec((1, l, cin), lambda i: (i, 0, 0)),
                  pl.BlockSpec((9, cin, 256), lambda i: (0, 0, 0)),
                  pl.BlockSpec((1, 256), lambda i: (0, 0)),
                  pl.BlockSpec((1, 256), lambda i: (0, 0)),
                  pl.BlockSpec((9, 256, 256), lambda i: (0, 0, 0)),
                  pl.BlockSpec((1, 256), lambda i: (0, 0)),
                  pl.BlockSpec((1, 256), lambda i: (0, 0)),
                  pl.BlockSpec((128, 1), lambda i: (0, 0)),
                  pl.BlockSpec((1, 1), lambda i: (0, 0))],
        out_specs=pl.BlockSpec((1, 1, 1), lambda i: (i, 0, 0)),
        scratch_shapes=[pltpu.VMEM((182, 256), jnp.bfloat16)],
        compiler_params=pltpu.CompilerParams(
            dimension_semantics=("parallel",)),
    )(xf, w4, s4, b4, w5, s5, b5, fw, fb)
    return out.reshape(n, 1)


def kernel(x,
           conv1_w, conv1_scale, conv1_bias,
           conv2_w, conv2_scale, conv2_bias,
           conv3_w, conv3_scale, conv3_bias,
           conv4_w, conv4_scale, conv4_bias,
           conv5_w, conv5_scale, conv5_bias,
           fc_w, fc_b):
    n = x.shape[0]
    w1 = _pack_w(_fold_haar(conv1_w, 4), 8)
    w2 = _pack_w(_fold_haar(conv2_w, 16), 2)
    w3 = _pack_w(_fold_haar(conv3_w, 64), 2)
    w4 = _pack_w(_fold_haar(conv4_w, 128), 2)
    w5 = _pack_w(conv5_w.astype(jnp.float32).reshape(9, 128, 128), 2)

    def rt(v, p):
        return jnp.tile(v.astype(jnp.float32).reshape(1, -1), (1, p))

    xf = _deint_pack_pad(x, 128, 128, 8)
    y1 = _conv_stage(xf, w1, rt(conv1_scale, 8), rt(conv1_bias, 8),
                     128, 18, 1152)
    xf = _deint_pack_pad(_unpack(y1, 128, 18, 8, 16), 64, 64, 2)
    y2 = _conv_stage(xf, w2, rt(conv2_scale, 2), rt(conv2_bias, 2),
                     64, 34, 1088)
    xf = _deint_pack_pad(_unpack(y2, 64, 34, 2, 64), 32, 32, 2)
    y3 = _conv_stage(xf, w3, rt(conv3_scale, 2), rt(conv3_bias, 2),
                     32, 18, 576)
    xf = _deint_pack_pad(_unpack(y3, 32, 18, 2, 128), 16, 16, 2)
    return _tail_stage(xf, w4, rt(conv4_scale, 2), rt(conv4_bias, 2),
                       w5, rt(conv5_scale, 2), rt(conv5_bias, 2),
                       fc_w.astype(jnp.float32).reshape(128, 1),
                       fc_b.astype(jnp.float32).reshape(1, 1))
